# baseline (speedup 1.0000x reference)
import functools

import jax
import jax.numpy as jnp
from jax import lax
from jax.experimental import pallas as pl
from jax.experimental.pallas import tpu as pltpu
from jax.experimental.pallas import tpu_sc as plsc

N = 10000
DIN = 128
H = 256
DOUT = 64
HH = H // 2
NC = 2
NS = 16
CH = 80
SEG = 2560
EPT = 20480
BLK = 400
EPS = 1e-16
NPAD = 10240


def _attn_aug(h, a_s, a_d):
    asrc = jnp.sum(h * a_s[None, :], axis=1, keepdims=True)
    adst = jnp.sum(h * a_d[None, :], axis=1, keepdims=True)
    return asrc, adst, jnp.stack([h[:, :HH], h[:, HH:]], axis=0)


def _layer1_tc_body(x_ref, w_ref, as_ref, ad_ref, haug_ref, asrc_ref, adst_ref):
    h = jnp.dot(x_ref[...], w_ref[...], preferred_element_type=jnp.float32)
    asrc, adst, haug = _attn_aug(h, as_ref[...], ad_ref[...])
    asrc_ref[...] = asrc
    adst_ref[...] = adst
    haug_ref[...] = haug


def _layer1_tc(x, w, a_s, a_d):
    grid = (N // BLK,)
    return pl.pallas_call(
        _layer1_tc_body,
        grid=grid,
        in_specs=[
            pl.BlockSpec((BLK, DIN), lambda i: (i, 0)),
            pl.BlockSpec((DIN, H), lambda i: (0, 0)),
            pl.BlockSpec((H,), lambda i: (0,)),
            pl.BlockSpec((H,), lambda i: (0,)),
        ],
        out_specs=[
            pl.BlockSpec((NC, BLK, HH), lambda i: (0, i, 0)),
            pl.BlockSpec((BLK, 1), lambda i: (i, 0)),
            pl.BlockSpec((BLK, 1), lambda i: (i, 0)),
        ],
        out_shape=[
            jax.ShapeDtypeStruct((NC, N, HH), jnp.float32),
            jax.ShapeDtypeStruct((N, 1), jnp.float32),
            jax.ShapeDtypeStruct((N, 1), jnp.float32),
        ],
    )(x, w, a_s, a_d)


def _norm_in(accA, accB, denA, denB, bias):
    x0 = accA / (denA + EPS)
    x1 = accB / (denB + EPS)
    return jax.nn.relu(jnp.concatenate([x0, x1], axis=1) + bias[None, :])


def _layer2_tc_body(accA_ref, accB_ref, denA_ref, denB_ref, bprev_ref,
                    w_ref, as_ref, ad_ref, haug_ref, asrc_ref, adst_ref):
    xin = _norm_in(accA_ref[...], accB_ref[...], denA_ref[...], denB_ref[...],
                   bprev_ref[...])
    h = jnp.dot(xin, w_ref[...], preferred_element_type=jnp.float32)
    asrc, adst, haug = _attn_aug(h, as_ref[...], ad_ref[...])
    asrc_ref[...] = asrc
    adst_ref[...] = adst
    haug_ref[...] = haug


def _layer2_tc(accA, accB, denA, denB, b_prev, w, a_s, a_d):
    grid = (N // BLK,)
    acc_spec = pl.BlockSpec((BLK, HH), lambda i: (i, 0))
    den_spec = pl.BlockSpec((BLK, 1), lambda i: (i, 0))
    return pl.pallas_call(
        _layer2_tc_body,
        grid=grid,
        in_specs=[
            acc_spec, acc_spec, den_spec, den_spec,
            pl.BlockSpec((H,), lambda i: (0,)),
            pl.BlockSpec((H, H), lambda i: (0, 0)),
            pl.BlockSpec((H,), lambda i: (0,)),
            pl.BlockSpec((H,), lambda i: (0,)),
        ],
        out_specs=[
            pl.BlockSpec((NC, BLK, HH), lambda i: (0, i, 0)),
            pl.BlockSpec((BLK, 1), lambda i: (i, 0)),
            pl.BlockSpec((BLK, 1), lambda i: (i, 0)),
        ],
        out_shape=[
            jax.ShapeDtypeStruct((NC, N, HH), jnp.float32),
            jax.ShapeDtypeStruct((N, 1), jnp.float32),
            jax.ShapeDtypeStruct((N, 1), jnp.float32),
        ],
    )(accA, accB, denA, denB, b_prev, w, a_s, a_d)


def _head_tc_body(xA_ref, xB_ref, xdA_ref, xdB_ref, yA_ref, yB_ref,
                  ydA_ref, ydB_ref, bx_ref, by_ref,
                  fcw_ref, fcb_ref, ow_ref, ob_ref, o_ref):
    xx = _norm_in(xA_ref[...], xB_ref[...], xdA_ref[...], xdB_ref[...],
                  bx_ref[...])
    xy = _norm_in(yA_ref[...], yB_ref[...], ydA_ref[...], ydB_ref[...],
                  by_ref[...])
    fcw = fcw_ref[...]
    o1 = jnp.dot(xx, fcw[:H], preferred_element_type=jnp.float32)
    o1 = o1 + jnp.dot(xy, fcw[H:], preferred_element_type=jnp.float32)
    o1 = jax.nn.relu(o1 + fcb_ref[...][None, :])
    o = jnp.dot(o1, ow_ref[...], preferred_element_type=jnp.float32)
    o = o + ob_ref[...][None, :]
    o_ref[...] = jnp.concatenate(
        [o, jnp.zeros((o.shape[0], 128 - DOUT), jnp.float32)], axis=1)


def _head_tc(xA, xB, xdA, xdB, yA, yB, ydA, ydB,
             b_x2, b_y2, fc_W, fc_b, out_W, out_b):
    grid = (N // BLK,)
    acc_spec = pl.BlockSpec((BLK, HH), lambda i: (i, 0))
    den_spec = pl.BlockSpec((BLK, 1), lambda i: (i, 0))
    return pl.pallas_call(
        _head_tc_body,
        grid=grid,
        in_specs=[
            acc_spec, acc_spec, den_spec, den_spec,
            acc_spec, acc_spec, den_spec, den_spec,
            pl.BlockSpec((H,), lambda i: (0,)),
            pl.BlockSpec((H,), lambda i: (0,)),
            pl.BlockSpec((2 * H, H), lambda i: (0, 0)),
            pl.BlockSpec((H,), lambda i: (0,)),
            pl.BlockSpec((H, DOUT), lambda i: (0, 0)),
            pl.BlockSpec((DOUT,), lambda i: (0,)),
        ],
        out_specs=pl.BlockSpec((BLK, 128), lambda i: (i, 0)),
        out_shape=jax.ShapeDtypeStruct((N, 128), jnp.float32),
    )(xA, xB, xdA, xdB, yA, yB, ydA, ydB,
      b_x2, b_y2, fc_W, fc_b, out_W, out_b)


@functools.cache
def _sc_mesh():
    return plsc.VectorSubcoreMesh(
        core_axis_name="c", subcore_axis_name="s",
        num_cores=NC, num_subcores=NS)


def _build_chunk(srcseg, dstseg, base, asr, adr, sidx, didx, coef, coffs):
    for g in range(CH // 16):
        sv = srcseg[pl.ds(base + g * 16, 16)]
        dv = dstseg[pl.ds(base + g * 16, 16)]
        sidx[pl.ds(g * 16, 16)] = sv + coffs
        didx[pl.ds(g * 16, 16)] = dv
        a1 = plsc.load_gather(asr, [sv])
        a2 = plsc.load_gather(adr, [dv])
        al = a1 + a2
        al = jnp.where(al >= 0.0, al, 0.2 * al)
        coef[pl.ds(g * 16, 16)] = jnp.exp(al)


def _scale_chunk(rows, coef):
    @pl.loop(0, CH // 16)
    def _scale(g):
        ev = coef[pl.ds(g * 16, 16)]
        for j in range(16):
            r = g * 16 + j
            e = ev[j]
            for w in range(HH // 16):
                rows[r, pl.ds(w * 16, 16)] = rows[r, pl.ds(w * 16, 16)] * e


def _mp_body(esrc, edst, haug, asrc, adst, out, den_out,
             segsA, segdA, asr, adr,
             rows, sidx, didx, coef, zvec,
             acc, den, sem_g):
    c = lax.axis_index("c")
    s = lax.axis_index("s")
    nsegs = EPT // SEG
    rows_per_tile = NPAD // NS

    pltpu.sync_copy(asrc, asr)
    pltpu.sync_copy(adst, adr)

    @pl.loop(0, CH)
    def _zr(i):
        for w in range(HH // 16):
            rows[i, pl.ds(w * 16, 16)] = jnp.zeros((16,), jnp.float32)

    @pl.loop(0, rows_per_tile // 16)
    def _zv(i):
        zvec[pl.ds(i * 16, 16)] = jnp.zeros((16,), jnp.float32)

    @pl.loop(0, rows_per_tile // CH)
    def _zc(i):
        pltpu.sync_copy(rows, acc.at[pl.ds(s * rows_per_tile + i * CH, CH)])

    pltpu.sync_copy(zvec, den.at[pl.ds(s * rows_per_tile, rows_per_tile)])

    plsc.subcore_barrier()

    coffs = (c * N).astype(jnp.int32)
    ebase = s * EPT

    @pl.loop(0, EPT // SEG)
    def _seg(t):
        off = ebase + t * SEG
        pltpu.sync_copy(esrc.at[pl.ds(off, SEG)], segsA)
        pltpu.sync_copy(edst.at[pl.ds(off, SEG)], segdA)

        @pl.loop(0, SEG // CH)
        def _chunk(i):
            _build_chunk(segsA, segdA, i * CH, asr, adr, sidx, didx, coef,
                         coffs)
            pltpu.async_copy(haug.at[sidx], rows, sem_g).wait()
            _scale_chunk(rows, coef)
            pltpu.sync_copy(rows, acc.at[didx], add=True)
            pltpu.sync_copy(coef, den.at[didx], add=True)

    plsc.subcore_barrier()

    pltpu.sync_copy(acc.at[pl.ds(s * rows_per_tile, rows_per_tile)],
                    out.at[c, pl.ds(s * rows_per_tile, rows_per_tile)])
    pltpu.sync_copy(den.at[pl.ds(s * rows_per_tile, rows_per_tile)],
                    den_out.at[pl.ds(c * NPAD + s * rows_per_tile, rows_per_tile)])


def _message_pass(edges, haug2, asrc_pad, adst_pad):
    f = pl.kernel(
        _mp_body,
        out_type=[
            jax.ShapeDtypeStruct((NC, NPAD, HH), jnp.float32),
            jax.ShapeDtypeStruct((NC * NPAD,), jnp.float32),
        ],
        mesh=_sc_mesh(),
        compiler_params=pltpu.CompilerParams(needs_layout_passes=False),
        scratch_types=[
            pltpu.VMEM((SEG,), jnp.int32),
            pltpu.VMEM((SEG,), jnp.int32),
            pltpu.VMEM((NPAD,), jnp.float32),
            pltpu.VMEM((NPAD,), jnp.float32),
            pltpu.VMEM((CH, HH), jnp.float32),
            pltpu.VMEM((CH,), jnp.int32),
            pltpu.VMEM((CH,), jnp.int32),
            pltpu.VMEM((CH,), jnp.float32),
            pltpu.VMEM((NPAD // NS,), jnp.float32),
            pltpu.VMEM_SHARED((NPAD, HH), jnp.float32),
            pltpu.VMEM_SHARED((NPAD,), jnp.float32),
            pltpu.SemaphoreType.DMA,
        ],
    )
    return f(edges[0], edges[1], haug2, asrc_pad, adst_pad)


def _gather_rows_body(o, idx, out, idxv, rowsv, sem):
    c = lax.axis_index("c")
    s = lax.axis_index("s")
    wid = s * NC + c
    per = 160
    base = wid * per
    pltpu.sync_copy(idx.at[pl.ds(base, per)], idxv)
    cp0 = pltpu.async_copy(o.at[idxv.at[pl.ds(0, 80)]], rowsv.at[pl.ds(0, 80)], sem)
    cp1 = pltpu.async_copy(o.at[idxv.at[pl.ds(80, 80)]], rowsv.at[pl.ds(80, 80)], sem)
    cp0.wait()
    cp1.wait()
    pltpu.sync_copy(rowsv, out.at[pl.ds(base, per)])


def _gather_rows(o, idxpad):
    B = idxpad.shape[0]
    f = pl.kernel(
        _gather_rows_body,
        out_type=jax.ShapeDtypeStruct((B, 128), jnp.float32),
        mesh=_sc_mesh(),
        compiler_params=pltpu.CompilerParams(needs_layout_passes=False),
        scratch_types=[
            pltpu.VMEM((160,), jnp.int32),
            pltpu.VMEM((160, 128), jnp.float32),
            pltpu.SemaphoreType.DMA,
        ],
    )
    return f(o, idxpad)


def kernel(x, edge_index_x, edge_index_y, indices,
           W_x1, as_x1, ad_x1, b_x1,
           W_x2, as_x2, ad_x2, b_x2,
           W_y1, as_y1, ad_y1, b_y1,
           W_y2, as_y2, ad_y2, b_y2,
           fc_W, fc_b, out_W, out_b):
    def pad_edges(e):
        e = e.astype(jnp.int32)
        extra = NS * EPT - e.shape[1]
        pad = jnp.stack([jnp.zeros((extra,), jnp.int32),
                         N + (jnp.arange(extra, dtype=jnp.int32) % (NPAD - N))],
                        axis=0)
        return jnp.concatenate([e, pad], axis=1)

    ex = pad_edges(edge_index_x)
    ey = pad_edges(edge_index_y)

    def sc_pass(edges, haug, asrc, adst):
        acc, den = _message_pass(edges, haug.reshape(NC * N, HH),
                                 jnp.pad(asrc.reshape(N), (0, NPAD - N)),
                                 jnp.pad(adst.reshape(N), (0, NPAD - N)))
        den = den.reshape(NC, NPAD)
        return (acc[0, :N], acc[1, :N],
                den[0, :N].reshape(N, 1), den[1, :N].reshape(N, 1))

    haug, asrc, adst = _layer1_tc(x, W_x1, as_x1, ad_x1)
    xA, xB, xdA, xdB = sc_pass(ex, haug, asrc, adst)
    haug, asrc, adst = _layer2_tc(xA, xB, xdA, xdB, b_x1, W_x2, as_x2, ad_x2)
    xA2, xB2, xdA2, xdB2 = sc_pass(ex, haug, asrc, adst)

    haug, asrc, adst = _layer1_tc(x, W_y1, as_y1, ad_y1)
    yA, yB, ydA, ydB = sc_pass(ey, haug, asrc, adst)
    haug, asrc, adst = _layer2_tc(yA, yB, ydA, ydB, b_y1, W_y2, as_y2, ad_y2)
    yA2, yB2, ydA2, ydB2 = sc_pass(ey, haug, asrc, adst)

    o = _head_tc(xA2, xB2, xdA2, xdB2, yA2, yB2, ydA2, ydB2,
                 b_x2, b_y2, fc_W, fc_b, out_W, out_b)

    nidx = indices.shape[0]
    npad = (-nidx) % (NC * NS * 160)
    idxpad = jnp.pad(indices.astype(jnp.int32), (0, npad))
    gathered = _gather_rows(o, idxpad)
    return gathered[:nidx, :DOUT]


# speedup vs baseline: 1.7403x; 1.7403x over previous
import functools

import jax
import jax.numpy as jnp
from jax import lax
from jax.experimental import pallas as pl
from jax.experimental.pallas import tpu as pltpu
from jax.experimental.pallas import tpu_sc as plsc

N = 10000
DIN = 128
H = 256
DOUT = 64
HH = H // 2
NC = 2
NS = 16
CH = 80
SEG = 2000
EPT = 20000
BLK = 400
EPS = 1e-16
NPAD = 10240


def _attn_aug(h, a_s, a_d):
    asrc = jnp.sum(h * a_s[None, :], axis=1, keepdims=True)
    adst = jnp.sum(h * a_d[None, :], axis=1, keepdims=True)
    return asrc, adst, jnp.stack([h[:, :HH], h[:, HH:]], axis=0)


def _layer1_tc_body(x_ref, w_ref, as_ref, ad_ref, haug_ref, asrc_ref, adst_ref):
    h = jnp.dot(x_ref[...], w_ref[...], preferred_element_type=jnp.float32)
    asrc, adst, haug = _attn_aug(h, as_ref[...], ad_ref[...])
    asrc_ref[...] = asrc
    adst_ref[...] = adst
    haug_ref[...] = haug


def _layer1_tc(x, w, a_s, a_d):
    grid = (N // BLK,)
    return pl.pallas_call(
        _layer1_tc_body,
        grid=grid,
        in_specs=[
            pl.BlockSpec((BLK, DIN), lambda i: (i, 0)),
            pl.BlockSpec((DIN, H), lambda i: (0, 0)),
            pl.BlockSpec((H,), lambda i: (0,)),
            pl.BlockSpec((H,), lambda i: (0,)),
        ],
        out_specs=[
            pl.BlockSpec((NC, BLK, HH), lambda i: (0, i, 0)),
            pl.BlockSpec((BLK, 1), lambda i: (i, 0)),
            pl.BlockSpec((BLK, 1), lambda i: (i, 0)),
        ],
        out_shape=[
            jax.ShapeDtypeStruct((NC, N, HH), jnp.float32),
            jax.ShapeDtypeStruct((N, 1), jnp.float32),
            jax.ShapeDtypeStruct((N, 1), jnp.float32),
        ],
    )(x, w, a_s, a_d)


def _norm_in(accA, accB, denA, denB, bias):
    x0 = accA / (denA + EPS)
    x1 = accB / (denB + EPS)
    return jax.nn.relu(jnp.concatenate([x0, x1], axis=1) + bias[None, :])


def _layer2_tc_body(accA_ref, accB_ref, denA_ref, denB_ref, bprev_ref,
                    w_ref, as_ref, ad_ref, haug_ref, asrc_ref, adst_ref):
    xin = _norm_in(accA_ref[...], accB_ref[...], denA_ref[...], denB_ref[...],
                   bprev_ref[...])
    h = jnp.dot(xin, w_ref[...], preferred_element_type=jnp.float32)
    asrc, adst, haug = _attn_aug(h, as_ref[...], ad_ref[...])
    asrc_ref[...] = asrc
    adst_ref[...] = adst
    haug_ref[...] = haug


def _layer2_tc(accA, accB, denA, denB, b_prev, w, a_s, a_d):
    grid = (N // BLK,)
    acc_spec = pl.BlockSpec((BLK, HH), lambda i: (i, 0))
    den_spec = pl.BlockSpec((BLK, 1), lambda i: (i, 0))
    return pl.pallas_call(
        _layer2_tc_body,
        grid=grid,
        in_specs=[
            acc_spec, acc_spec, den_spec, den_spec,
            pl.BlockSpec((H,), lambda i: (0,)),
            pl.BlockSpec((H, H), lambda i: (0, 0)),
            pl.BlockSpec((H,), lambda i: (0,)),
            pl.BlockSpec((H,), lambda i: (0,)),
        ],
        out_specs=[
            pl.BlockSpec((NC, BLK, HH), lambda i: (0, i, 0)),
            pl.BlockSpec((BLK, 1), lambda i: (i, 0)),
            pl.BlockSpec((BLK, 1), lambda i: (i, 0)),
        ],
        out_shape=[
            jax.ShapeDtypeStruct((NC, N, HH), jnp.float32),
            jax.ShapeDtypeStruct((N, 1), jnp.float32),
            jax.ShapeDtypeStruct((N, 1), jnp.float32),
        ],
    )(accA, accB, denA, denB, b_prev, w, a_s, a_d)


def _head_tc_body(xA_ref, xB_ref, xdA_ref, xdB_ref, yA_ref, yB_ref,
                  ydA_ref, ydB_ref, bx_ref, by_ref,
                  fcw_ref, fcb_ref, ow_ref, ob_ref, o_ref):
    xx = _norm_in(xA_ref[...], xB_ref[...], xdA_ref[...], xdB_ref[...],
                  bx_ref[...])
    xy = _norm_in(yA_ref[...], yB_ref[...], ydA_ref[...], ydB_ref[...],
                  by_ref[...])
    fcw = fcw_ref[...]
    o1 = jnp.dot(xx, fcw[:H], preferred_element_type=jnp.float32)
    o1 = o1 + jnp.dot(xy, fcw[H:], preferred_element_type=jnp.float32)
    o1 = jax.nn.relu(o1 + fcb_ref[...][None, :])
    o = jnp.dot(o1, ow_ref[...], preferred_element_type=jnp.float32)
    o = o + ob_ref[...][None, :]
    o_ref[...] = jnp.concatenate(
        [o, jnp.zeros((o.shape[0], 128 - DOUT), jnp.float32)], axis=1)


def _head_tc(xA, xB, xdA, xdB, yA, yB, ydA, ydB,
             b_x2, b_y2, fc_W, fc_b, out_W, out_b):
    grid = (N // BLK,)
    acc_spec = pl.BlockSpec((BLK, HH), lambda i: (i, 0))
    den_spec = pl.BlockSpec((BLK, 1), lambda i: (i, 0))
    return pl.pallas_call(
        _head_tc_body,
        grid=grid,
        in_specs=[
            acc_spec, acc_spec, den_spec, den_spec,
            acc_spec, acc_spec, den_spec, den_spec,
            pl.BlockSpec((H,), lambda i: (0,)),
            pl.BlockSpec((H,), lambda i: (0,)),
            pl.BlockSpec((2 * H, H), lambda i: (0, 0)),
            pl.BlockSpec((H,), lambda i: (0,)),
            pl.BlockSpec((H, DOUT), lambda i: (0, 0)),
            pl.BlockSpec((DOUT,), lambda i: (0,)),
        ],
        out_specs=pl.BlockSpec((BLK, 128), lambda i: (i, 0)),
        out_shape=jax.ShapeDtypeStruct((N, 128), jnp.float32),
    )(xA, xB, xdA, xdB, yA, yB, ydA, ydB,
      b_x2, b_y2, fc_W, fc_b, out_W, out_b)


@functools.cache
def _sc_mesh():
    return plsc.VectorSubcoreMesh(
        core_axis_name="c", subcore_axis_name="s",
        num_cores=NC, num_subcores=NS)


def _build_chunk(srcseg, dstseg, base, asr, adr, sidx, didx, coef, coffs):
    for g in range(CH // 16):
        sv = srcseg[pl.ds(base + g * 16, 16)]
        dv = dstseg[pl.ds(base + g * 16, 16)]
        sidx[pl.ds(g * 16, 16)] = sv + coffs
        didx[pl.ds(g * 16, 16)] = dv
        a1 = plsc.load_gather(asr, [sv])
        a2 = plsc.load_gather(adr, [dv])
        al = a1 + a2
        al = jnp.where(al >= 0.0, al, 0.2 * al)
        coef[pl.ds(g * 16, 16)] = jnp.exp(al)


def _scale_chunk(rows, coef):
    @pl.loop(0, CH // 16)
    def _scale(g):
        ev = coef[pl.ds(g * 16, 16)]
        for j in range(16):
            r = g * 16 + j
            e = ev[j]
            for w in range(HH // 16):
                rows[r, pl.ds(w * 16, 16)] = rows[r, pl.ds(w * 16, 16)] * e


def _mp_body(esrc, edst, haug, asrc, adst, out, den_out,
             segsA, segdA, asr, adr,
             rows, sidx, didx, coef, zvec,
             acc, den, sem_g):
    c = lax.axis_index("c")
    s = lax.axis_index("s")
    nsegs = EPT // SEG
    rows_per_tile = NPAD // NS

    pltpu.sync_copy(asrc, asr)
    pltpu.sync_copy(adst, adr)

    @pl.loop(0, CH)
    def _zr(i):
        for w in range(HH // 16):
            rows[i, pl.ds(w * 16, 16)] = jnp.zeros((16,), jnp.float32)

    @pl.loop(0, rows_per_tile // 16)
    def _zv(i):
        zvec[pl.ds(i * 16, 16)] = jnp.zeros((16,), jnp.float32)

    @pl.loop(0, rows_per_tile // CH)
    def _zc(i):
        pltpu.sync_copy(rows, acc.at[pl.ds(s * rows_per_tile + i * CH, CH)])

    pltpu.sync_copy(zvec, den.at[pl.ds(s * rows_per_tile, rows_per_tile)])

    plsc.subcore_barrier()

    coffs = (c * N).astype(jnp.int32)
    ebase = s * EPT

    @pl.loop(0, EPT // SEG)
    def _seg(t):
        off = ebase + t * SEG
        pltpu.sync_copy(esrc.at[pl.ds(off, SEG)], segsA)
        pltpu.sync_copy(edst.at[pl.ds(off, SEG)], segdA)

        @pl.loop(0, SEG // CH)
        def _chunk(i):
            _build_chunk(segsA, segdA, i * CH, asr, adr, sidx, didx, coef,
                         coffs)
            pltpu.async_copy(haug.at[sidx], rows, sem_g).wait()
            _scale_chunk(rows, coef)
            pltpu.sync_copy(rows, acc.at[didx], add=True)
            pltpu.sync_copy(coef, den.at[didx], add=True)

    plsc.subcore_barrier()

    pltpu.sync_copy(acc.at[pl.ds(s * rows_per_tile, rows_per_tile)],
                    out.at[c, pl.ds(s * rows_per_tile, rows_per_tile)])
    pltpu.sync_copy(den.at[pl.ds(s * rows_per_tile, rows_per_tile)],
                    den_out.at[pl.ds(c * NPAD + s * rows_per_tile, rows_per_tile)])


def _message_pass(edges, haug2, asrc_pad, adst_pad):
    f = pl.kernel(
        _mp_body,
        out_type=[
            jax.ShapeDtypeStruct((NC, NPAD, HH), jnp.float32),
            jax.ShapeDtypeStruct((NC * NPAD,), jnp.float32),
        ],
        mesh=_sc_mesh(),
        compiler_params=pltpu.CompilerParams(needs_layout_passes=False),
        scratch_types=[
            pltpu.VMEM((SEG,), jnp.int32),
            pltpu.VMEM((SEG,), jnp.int32),
            pltpu.VMEM((NPAD,), jnp.float32),
            pltpu.VMEM((NPAD,), jnp.float32),
            pltpu.VMEM((CH, HH), jnp.float32),
            pltpu.VMEM((CH,), jnp.int32),
            pltpu.VMEM((CH,), jnp.int32),
            pltpu.VMEM((CH,), jnp.float32),
            pltpu.VMEM((NPAD // NS,), jnp.float32),
            pltpu.VMEM_SHARED((NPAD, HH), jnp.float32),
            pltpu.VMEM_SHARED((NPAD,), jnp.float32),
            pltpu.SemaphoreType.DMA,
        ],
    )
    return f(edges[0], edges[1], haug2, asrc_pad, adst_pad)


def _gather_rows_body(o, idx, out, idxv, rowsv, sem):
    c = lax.axis_index("c")
    s = lax.axis_index("s")
    wid = s * NC + c
    per = 160
    base = wid * per
    pltpu.sync_copy(idx.at[pl.ds(base, per)], idxv)
    cp0 = pltpu.async_copy(o.at[idxv.at[pl.ds(0, 80)]], rowsv.at[pl.ds(0, 80)], sem)
    cp1 = pltpu.async_copy(o.at[idxv.at[pl.ds(80, 80)]], rowsv.at[pl.ds(80, 80)], sem)
    cp0.wait()
    cp1.wait()
    pltpu.sync_copy(rowsv, out.at[pl.ds(base, per)])


def _gather_rows(o, idxpad):
    B = idxpad.shape[0]
    f = pl.kernel(
        _gather_rows_body,
        out_type=jax.ShapeDtypeStruct((B, 128), jnp.float32),
        mesh=_sc_mesh(),
        compiler_params=pltpu.CompilerParams(needs_layout_passes=False),
        scratch_types=[
            pltpu.VMEM((160,), jnp.int32),
            pltpu.VMEM((160, 128), jnp.float32),
            pltpu.SemaphoreType.DMA,
        ],
    )
    return f(o, idxpad)


def kernel(x, edge_index_x, edge_index_y, indices,
           W_x1, as_x1, ad_x1, b_x1,
           W_x2, as_x2, ad_x2, b_x2,
           W_y1, as_y1, ad_y1, b_y1,
           W_y2, as_y2, ad_y2, b_y2,
           fc_W, fc_b, out_W, out_b):
    ex = edge_index_x.astype(jnp.int32)
    ey = edge_index_y.astype(jnp.int32)

    def sc_pass(edges, haug, asrc, adst):
        acc, den = _message_pass(edges, haug.reshape(NC * N, HH),
                                 jnp.pad(asrc.reshape(N), (0, NPAD - N)),
                                 jnp.pad(adst.reshape(N), (0, NPAD - N)))
        den = den.reshape(NC, NPAD)
        return (acc[0, :N], acc[1, :N],
                den[0, :N].reshape(N, 1), den[1, :N].reshape(N, 1))

    haug, asrc, adst = _layer1_tc(x, W_x1, as_x1, ad_x1)
    xA, xB, xdA, xdB = sc_pass(ex, haug, asrc, adst)
    haug, asrc, adst = _layer2_tc(xA, xB, xdA, xdB, b_x1, W_x2, as_x2, ad_x2)
    xA2, xB2, xdA2, xdB2 = sc_pass(ex, haug, asrc, adst)

    haug, asrc, adst = _layer1_tc(x, W_y1, as_y1, ad_y1)
    yA, yB, ydA, ydB = sc_pass(ey, haug, asrc, adst)
    haug, asrc, adst = _layer2_tc(yA, yB, ydA, ydB, b_y1, W_y2, as_y2, ad_y2)
    yA2, yB2, ydA2, ydB2 = sc_pass(ey, haug, asrc, adst)

    o = _head_tc(xA2, xB2, xdA2, xdB2, yA2, yB2, ydA2, ydB2,
                 b_x2, b_y2, fc_W, fc_b, out_W, out_b)

    nidx = indices.shape[0]
    npad = (-nidx) % (NC * NS * 160)
    idxpad = jnp.pad(indices.astype(jnp.int32), (0, npad))
    gathered = _gather_rows(o, idxpad)
    return gathered[:nidx, :DOUT]


# speedup vs baseline: 3.0100x; 1.7296x over previous
import functools

import jax
import jax.numpy as jnp
from jax import lax
from jax.experimental import pallas as pl
from jax.experimental.pallas import tpu as pltpu
from jax.experimental.pallas import tpu_sc as plsc

N = 10000
DIN = 128
H = 256
DOUT = 64
HH = H // 2
NC = 2
NS = 16
CH = 80
SEG = 2000
EPT = 20000
BLK = 400
EPS = 1e-16
NPAD = 10240


def _attn_aug(h, a_s, a_d):
    asrc = jnp.sum(h * a_s[None, :], axis=1, keepdims=True)
    adst = jnp.sum(h * a_d[None, :], axis=1, keepdims=True)
    return asrc, adst, jnp.stack([h[:, :HH], h[:, HH:]], axis=0)


def _layer1_tc_body(x_ref, w_ref, as_ref, ad_ref, haug_ref, asrc_ref, adst_ref):
    h = jnp.dot(x_ref[...], w_ref[...], preferred_element_type=jnp.float32)
    asrc, adst, haug = _attn_aug(h, as_ref[...], ad_ref[...])
    asrc_ref[...] = asrc
    adst_ref[...] = adst
    haug_ref[...] = haug


def _layer1_tc(x, w, a_s, a_d):
    grid = (N // BLK,)
    return pl.pallas_call(
        _layer1_tc_body,
        grid=grid,
        in_specs=[
            pl.BlockSpec((BLK, DIN), lambda i: (i, 0)),
            pl.BlockSpec((DIN, H), lambda i: (0, 0)),
            pl.BlockSpec((H,), lambda i: (0,)),
            pl.BlockSpec((H,), lambda i: (0,)),
        ],
        out_specs=[
            pl.BlockSpec((NC, BLK, HH), lambda i: (0, i, 0)),
            pl.BlockSpec((BLK, 1), lambda i: (i, 0)),
            pl.BlockSpec((BLK, 1), lambda i: (i, 0)),
        ],
        out_shape=[
            jax.ShapeDtypeStruct((NC, N, HH), jnp.float32),
            jax.ShapeDtypeStruct((N, 1), jnp.float32),
            jax.ShapeDtypeStruct((N, 1), jnp.float32),
        ],
    )(x, w, a_s, a_d)


def _norm_in(accA, accB, denA, denB, bias):
    x0 = accA / (denA + EPS)
    x1 = accB / (denB + EPS)
    return jax.nn.relu(jnp.concatenate([x0, x1], axis=1) + bias[None, :])


def _layer2_tc_body(accA_ref, accB_ref, denA_ref, denB_ref, bprev_ref,
                    w_ref, as_ref, ad_ref, haug_ref, asrc_ref, adst_ref):
    xin = _norm_in(accA_ref[...], accB_ref[...], denA_ref[...], denB_ref[...],
                   bprev_ref[...])
    h = jnp.dot(xin, w_ref[...], preferred_element_type=jnp.float32)
    asrc, adst, haug = _attn_aug(h, as_ref[...], ad_ref[...])
    asrc_ref[...] = asrc
    adst_ref[...] = adst
    haug_ref[...] = haug


def _layer2_tc(accA, accB, denA, denB, b_prev, w, a_s, a_d):
    grid = (N // BLK,)
    acc_spec = pl.BlockSpec((BLK, HH), lambda i: (i, 0))
    den_spec = pl.BlockSpec((BLK, 1), lambda i: (i, 0))
    return pl.pallas_call(
        _layer2_tc_body,
        grid=grid,
        in_specs=[
            acc_spec, acc_spec, den_spec, den_spec,
            pl.BlockSpec((H,), lambda i: (0,)),
            pl.BlockSpec((H, H), lambda i: (0, 0)),
            pl.BlockSpec((H,), lambda i: (0,)),
            pl.BlockSpec((H,), lambda i: (0,)),
        ],
        out_specs=[
            pl.BlockSpec((NC, BLK, HH), lambda i: (0, i, 0)),
            pl.BlockSpec((BLK, 1), lambda i: (i, 0)),
            pl.BlockSpec((BLK, 1), lambda i: (i, 0)),
        ],
        out_shape=[
            jax.ShapeDtypeStruct((NC, N, HH), jnp.float32),
            jax.ShapeDtypeStruct((N, 1), jnp.float32),
            jax.ShapeDtypeStruct((N, 1), jnp.float32),
        ],
    )(accA, accB, denA, denB, b_prev, w, a_s, a_d)


def _head_tc_body(xA_ref, xB_ref, xdA_ref, xdB_ref, yA_ref, yB_ref,
                  ydA_ref, ydB_ref, bx_ref, by_ref,
                  fcw_ref, fcb_ref, ow_ref, ob_ref, o_ref):
    xx = _norm_in(xA_ref[...], xB_ref[...], xdA_ref[...], xdB_ref[...],
                  bx_ref[...])
    xy = _norm_in(yA_ref[...], yB_ref[...], ydA_ref[...], ydB_ref[...],
                  by_ref[...])
    fcw = fcw_ref[...]
    o1 = jnp.dot(xx, fcw[:H], preferred_element_type=jnp.float32)
    o1 = o1 + jnp.dot(xy, fcw[H:], preferred_element_type=jnp.float32)
    o1 = jax.nn.relu(o1 + fcb_ref[...][None, :])
    o = jnp.dot(o1, ow_ref[...], preferred_element_type=jnp.float32)
    o = o + ob_ref[...][None, :]
    o_ref[...] = jnp.concatenate(
        [o, jnp.zeros((o.shape[0], 128 - DOUT), jnp.float32)], axis=1)


def _head_tc(xA, xB, xdA, xdB, yA, yB, ydA, ydB,
             b_x2, b_y2, fc_W, fc_b, out_W, out_b):
    grid = (N // BLK,)
    acc_spec = pl.BlockSpec((BLK, HH), lambda i: (i, 0))
    den_spec = pl.BlockSpec((BLK, 1), lambda i: (i, 0))
    return pl.pallas_call(
        _head_tc_body,
        grid=grid,
        in_specs=[
            acc_spec, acc_spec, den_spec, den_spec,
            acc_spec, acc_spec, den_spec, den_spec,
            pl.BlockSpec((H,), lambda i: (0,)),
            pl.BlockSpec((H,), lambda i: (0,)),
            pl.BlockSpec((2 * H, H), lambda i: (0, 0)),
            pl.BlockSpec((H,), lambda i: (0,)),
            pl.BlockSpec((H, DOUT), lambda i: (0, 0)),
            pl.BlockSpec((DOUT,), lambda i: (0,)),
        ],
        out_specs=pl.BlockSpec((BLK, 128), lambda i: (i, 0)),
        out_shape=jax.ShapeDtypeStruct((N, 128), jnp.float32),
    )(xA, xB, xdA, xdB, yA, yB, ydA, ydB,
      b_x2, b_y2, fc_W, fc_b, out_W, out_b)


@functools.cache
def _sc_mesh():
    return plsc.VectorSubcoreMesh(
        core_axis_name="c", subcore_axis_name="s",
        num_cores=NC, num_subcores=NS)


def _build_chunk(srcseg, dstseg, base, asr, adr, sidx, didx, coef, coffs):
    for g in range(CH // 16):
        sv = srcseg[pl.ds(base + g * 16, 16)]
        dv = dstseg[pl.ds(base + g * 16, 16)]
        sidx[pl.ds(g * 16, 16)] = sv + coffs
        didx[pl.ds(g * 16, 16)] = dv
        a1 = plsc.load_gather(asr, [sv])
        a2 = plsc.load_gather(adr, [dv])
        al = a1 + a2
        al = jnp.where(al >= 0.0, al, 0.2 * al)
        coef[pl.ds(g * 16, 16)] = jnp.exp(al)


def _scale_chunk(rows, coef):
    @pl.loop(0, CH // 16)
    def _scale(g):
        ev = coef[pl.ds(g * 16, 16)]
        for j in range(16):
            r = g * 16 + j
            e = ev[j]
            for w in range(HH // 16):
                rows[r, pl.ds(w * 16, 16)] = rows[r, pl.ds(w * 16, 16)] * e


def _mp_body(esrc, edst, haug, asrc, adst, out, den_out,
             asr, adr, rowsA, rowsB, srcA, dstA, srcB, dstB,
             sidxA, didxA, coefA, sidxB, didxB, coefB, zvec,
             acc, den, sem_eA, sem_eB, sem_gA, sem_gB, sem_s, sem_d):
    c = lax.axis_index("c")
    s = lax.axis_index("s")
    rows_per_tile = NPAD // NS
    nchunks = EPT // CH
    pairs = nchunks // 2

    pltpu.sync_copy(asrc, asr)
    pltpu.sync_copy(adst, adr)

    @pl.loop(0, CH)
    def _zr(i):
        for w in range(HH // 16):
            rowsA[i, pl.ds(w * 16, 16)] = jnp.zeros((16,), jnp.float32)

    @pl.loop(0, rows_per_tile // 16)
    def _zv(i):
        zvec[pl.ds(i * 16, 16)] = jnp.zeros((16,), jnp.float32)

    @pl.loop(0, rows_per_tile // CH)
    def _zc(i):
        pltpu.sync_copy(rowsA, acc.at[pl.ds(s * rows_per_tile + i * CH, CH)])

    pltpu.sync_copy(zvec, den.at[pl.ds(s * rows_per_tile, rows_per_tile)])

    plsc.subcore_barrier()

    coffs = (c * N).astype(jnp.int32)
    ebase = s * EPT
    lastc = nchunks - 2

    bufA = (rowsA, srcA, dstA, sidxA, didxA, coefA, sem_eA, sem_gA)
    bufB = (rowsB, srcB, dstB, sidxB, didxB, coefB, sem_eB, sem_gB)

    def fetch(buf, chunk):
        rows, srcs, dsts, sidx, didx, coef, sem_e, sem_g = buf
        off = ebase + chunk * CH
        pltpu.async_copy(esrc.at[pl.ds(off, CH)], srcs, sem_e)
        pltpu.async_copy(edst.at[pl.ds(off, CH)], dsts, sem_e)

    def fetch_wait(buf):
        rows, srcs, dsts, sidx, didx, coef, sem_e, sem_g = buf
        pltpu.make_async_copy(esrc.at[pl.ds(0, CH)], srcs, sem_e).wait()
        pltpu.make_async_copy(edst.at[pl.ds(0, CH)], dsts, sem_e).wait()

    def build_and_gather(buf):
        rows, srcs, dsts, sidx, didx, coef, sem_e, sem_g = buf
        _build_chunk(srcs, dsts, 0, asr, adr, sidx, didx, coef, coffs)
        pltpu.async_copy(haug.at[sidx], rows, sem_g)

    def gather_wait(buf):
        rows, srcs, dsts, sidx, didx, coef, sem_e, sem_g = buf
        pltpu.make_async_copy(haug.at[sidx], rows, sem_g).wait()

    fetch(bufA, 0)
    fetch(bufB, 1)
    fetch_wait(bufA)
    build_and_gather(bufA)
    fetch(bufA, 2)
    fetch_wait(bufB)
    build_and_gather(bufB)
    fetch(bufB, 3)

    @pl.loop(0, pairs)
    def _pair(g):
        scat = []
        for buf in (bufA, bufB):
            rows, srcs, dsts, sidx, didx, coef, sem_e, sem_g = buf
            gather_wait(buf)
            _scale_chunk(rows, coef)
            d1 = pltpu.async_copy(rows, acc.at[didx], sem_s, add=True)
            d2 = pltpu.async_copy(coef, den.at[didx], sem_d, add=True)
            scat.append((d1, d2))
        for p, buf in enumerate((bufA, bufB)):
            d1, d2 = scat[p]
            d1.wait()
            d2.wait()
            fetch_wait(buf)
            build_and_gather(buf)
            nxt = jnp.minimum(2 * g + 4 + p, lastc + p)
            fetch(buf, nxt)

    gather_wait(bufA)
    gather_wait(bufB)
    fetch_wait(bufA)
    fetch_wait(bufB)

    plsc.subcore_barrier()

    pltpu.sync_copy(acc.at[pl.ds(s * rows_per_tile, rows_per_tile)],
                    out.at[c, pl.ds(s * rows_per_tile, rows_per_tile)])
    pltpu.sync_copy(den.at[pl.ds(s * rows_per_tile, rows_per_tile)],
                    den_out.at[pl.ds(c * NPAD + s * rows_per_tile, rows_per_tile)])


def _message_pass(edges, haug2, asrc_pad, adst_pad):
    f = pl.kernel(
        _mp_body,
        out_type=[
            jax.ShapeDtypeStruct((NC, NPAD, HH), jnp.float32),
            jax.ShapeDtypeStruct((NC * NPAD,), jnp.float32),
        ],
        mesh=_sc_mesh(),
        compiler_params=pltpu.CompilerParams(needs_layout_passes=False),
        scratch_types=[
            pltpu.VMEM((N,), jnp.float32),
            pltpu.VMEM((N,), jnp.float32),
            pltpu.VMEM((CH, HH), jnp.float32),
            pltpu.VMEM((CH, HH), jnp.float32),
            pltpu.VMEM((CH,), jnp.int32),
            pltpu.VMEM((CH,), jnp.int32),
            pltpu.VMEM((CH,), jnp.int32),
            pltpu.VMEM((CH,), jnp.int32),
            pltpu.VMEM((CH,), jnp.int32),
            pltpu.VMEM((CH,), jnp.int32),
            pltpu.VMEM((CH,), jnp.float32),
            pltpu.VMEM((CH,), jnp.int32),
            pltpu.VMEM((CH,), jnp.int32),
            pltpu.VMEM((CH,), jnp.float32),
            pltpu.VMEM((NPAD // NS,), jnp.float32),
            pltpu.VMEM_SHARED((NPAD, HH), jnp.float32),
            pltpu.VMEM_SHARED((NPAD,), jnp.float32),
            pltpu.SemaphoreType.DMA,
            pltpu.SemaphoreType.DMA,
            pltpu.SemaphoreType.DMA,
            pltpu.SemaphoreType.DMA,
            pltpu.SemaphoreType.DMA,
            pltpu.SemaphoreType.DMA,
        ],
    )
    return f(edges[0], edges[1], haug2, asrc_pad, adst_pad)


def _gather_rows_body(o, idx, out, idxv, rowsv, sem):
    c = lax.axis_index("c")
    s = lax.axis_index("s")
    wid = s * NC + c
    per = 160
    base = wid * per
    pltpu.sync_copy(idx.at[pl.ds(base, per)], idxv)
    cp0 = pltpu.async_copy(o.at[idxv.at[pl.ds(0, 80)]], rowsv.at[pl.ds(0, 80)], sem)
    cp1 = pltpu.async_copy(o.at[idxv.at[pl.ds(80, 80)]], rowsv.at[pl.ds(80, 80)], sem)
    cp0.wait()
    cp1.wait()
    pltpu.sync_copy(rowsv, out.at[pl.ds(base, per)])


def _gather_rows(o, idxpad):
    B = idxpad.shape[0]
    f = pl.kernel(
        _gather_rows_body,
        out_type=jax.ShapeDtypeStruct((B, 128), jnp.float32),
        mesh=_sc_mesh(),
        compiler_params=pltpu.CompilerParams(needs_layout_passes=False),
        scratch_types=[
            pltpu.VMEM((160,), jnp.int32),
            pltpu.VMEM((160, 128), jnp.float32),
            pltpu.SemaphoreType.DMA,
        ],
    )
    return f(o, idxpad)


def kernel(x, edge_index_x, edge_index_y, indices,
           W_x1, as_x1, ad_x1, b_x1,
           W_x2, as_x2, ad_x2, b_x2,
           W_y1, as_y1, ad_y1, b_y1,
           W_y2, as_y2, ad_y2, b_y2,
           fc_W, fc_b, out_W, out_b):
    ex = edge_index_x.astype(jnp.int32)
    ey = edge_index_y.astype(jnp.int32)

    def sc_pass(edges, haug, asrc, adst):
        acc, den = _message_pass(edges, haug.reshape(NC * N, HH),
                                 asrc.reshape(N), adst.reshape(N))
        den = den.reshape(NC, NPAD)
        return (acc[0, :N], acc[1, :N],
                den[0, :N].reshape(N, 1), den[1, :N].reshape(N, 1))

    haug, asrc, adst = _layer1_tc(x, W_x1, as_x1, ad_x1)
    xA, xB, xdA, xdB = sc_pass(ex, haug, asrc, adst)
    haug, asrc, adst = _layer2_tc(xA, xB, xdA, xdB, b_x1, W_x2, as_x2, ad_x2)
    xA2, xB2, xdA2, xdB2 = sc_pass(ex, haug, asrc, adst)

    haug, asrc, adst = _layer1_tc(x, W_y1, as_y1, ad_y1)
    yA, yB, ydA, ydB = sc_pass(ey, haug, asrc, adst)
    haug, asrc, adst = _layer2_tc(yA, yB, ydA, ydB, b_y1, W_y2, as_y2, ad_y2)
    yA2, yB2, ydA2, ydB2 = sc_pass(ey, haug, asrc, adst)

    o = _head_tc(xA2, xB2, xdA2, xdB2, yA2, yB2, ydA2, ydB2,
                 b_x2, b_y2, fc_W, fc_b, out_W, out_b)

    nidx = indices.shape[0]
    npad = (-nidx) % (NC * NS * 160)
    idxpad = jnp.pad(indices.astype(jnp.int32), (0, npad))
    gathered = _gather_rows(o, idxpad)
    return gathered[:nidx, :DOUT]
